# baseline (speedup 1.0000x reference)
import functools

import jax
import jax.numpy as jnp
from jax import lax
from jax.experimental import pallas as pl
from jax.experimental.pallas import tpu as pltpu
from jax.experimental.pallas import tpu_sc as plsc

_BATCH, _SEQ, _D = 16384, 20, 64
_B = _BATCH * _SEQ
_NC, _NS = 2, 16
_NW = _NC * _NS
_BPW = _B // _NW
_G = 128
_NG = _BPW // _G


def _make_lookup():
    mesh = plsc.VectorSubcoreMesh(core_axis_name="c", subcore_axis_name="s")

    @functools.partial(
        pl.kernel,
        mesh=mesh,
        out_type=jax.ShapeDtypeStruct((_B, _D), jnp.float32),
        scratch_types=[
            pltpu.VMEM((_NG, _G), jnp.int32),
            pltpu.VMEM((_G, _D), jnp.float32),
            pltpu.SemaphoreType.DMA,
        ],
        compiler_params=pltpu.CompilerParams(use_tc_tiling_on_sc=False),
    )
    def lookup(ids_hbm, table_hbm, out_hbm, idx_v, rows_v, sem):
        wid = lax.axis_index("s") * _NC + lax.axis_index("c")
        pltpu.sync_copy(ids_hbm.at[wid], idx_v)

        def body(g, carry):
            pltpu.async_copy(table_hbm.at[idx_v.at[g]], rows_v, sem).wait()
            pltpu.sync_copy(rows_v, out_hbm.at[pl.ds(wid * _BPW + g * _G, _G)])
            return carry

        lax.fori_loop(0, _NG, body, 0)

    return lookup


_lookup = _make_lookup()


def kernel(input_ids, W):
    ids = input_ids.reshape(_NW, _NG, _G).astype(jnp.int32)
    out = _lookup(ids, W)
    return out.reshape(_BATCH, _SEQ, _D)


# speedup vs baseline: 1.0645x; 1.0645x over previous
import functools

import jax
import jax.numpy as jnp
from jax import lax
from jax.experimental import pallas as pl
from jax.experimental.pallas import tpu as pltpu
from jax.experimental.pallas import tpu_sc as plsc

_BATCH, _SEQ, _D = 16384, 20, 64
_B = _BATCH * _SEQ
_NC, _NS = 2, 16
_NW = _NC * _NS
_BPW = _B // _NW
_G = 128
_NG = _BPW // _G
_NBUF = 8
_K = 4
_NT = _NG // _NBUF


def _make_lookup():
    mesh = plsc.VectorSubcoreMesh(core_axis_name="c", subcore_axis_name="s")

    @functools.partial(
        pl.kernel,
        mesh=mesh,
        out_type=jax.ShapeDtypeStruct((_B, _D), jnp.float32),
        scratch_types=[
            pltpu.VMEM((_NG, _G), jnp.int32),
            pltpu.VMEM((_NBUF, _G, _D), jnp.float32),
            pltpu.SemaphoreType.DMA((_NBUF,)),
            pltpu.SemaphoreType.DMA((_NBUF,)),
        ],
        compiler_params=pltpu.CompilerParams(use_tc_tiling_on_sc=False),
    )
    def lookup(ids_hbm, table_hbm, out_hbm, idx_v, rows_v, gsem, osem):
        wid = lax.axis_index("s") * _NC + lax.axis_index("c")
        base = wid * _BPW
        pltpu.sync_copy(ids_hbm.at[wid], idx_v)

        def start_gather(g, b):
            pltpu.make_async_copy(
                table_hbm.at[idx_v.at[g]], rows_v.at[b], gsem.at[b]).start()

        def wait_gather(b):
            pltpu.make_async_copy(
                table_hbm.at[pl.ds(0, _G)], rows_v.at[b], gsem.at[b]).wait()

        def start_out(g, b):
            pltpu.make_async_copy(
                rows_v.at[b], out_hbm.at[pl.ds(base + g * _G, _G)],
                osem.at[b]).start()

        def wait_out(b):
            pltpu.make_async_copy(
                rows_v.at[b], out_hbm.at[pl.ds(base, _G)], osem.at[b]).wait()

        for b in range(_K):
            start_gather(b, b)

        for b in range(_NBUF):
            wait_gather(b)
            start_out(b, b)
            s4 = (b + _K) % _NBUF
            if b < _K:
                start_gather(b + _K, s4)
            else:
                wait_out(s4)
                start_gather(b + _K, s4)

        def revolution(t, carry):
            for b in range(_NBUF):
                g = t * _NBUF + b
                wait_gather(b)
                start_out(g, b)
                s4 = (b + _K) % _NBUF
                wait_out(s4)
                start_gather(g + _K, s4)
            return carry

        lax.fori_loop(1, _NT - 1, revolution, 0)

        for b in range(_NBUF):
            g = (_NT - 1) * _NBUF + b
            wait_gather(b)
            start_out(g, b)
            if b < _K:
                s4 = (b + _K) % _NBUF
                wait_out(s4)
                start_gather(g + _K, s4)

        for b in range(_NBUF):
            wait_out(b)

    return lookup


_lookup = _make_lookup()


def kernel(input_ids, W):
    ids = input_ids.reshape(_NW, _NG, _G).astype(jnp.int32)
    out = _lookup(ids, W)
    return out.reshape(_BATCH, _SEQ, _D)
